# baseline (speedup 1.0000x reference)
import functools

import numpy as np

import jax
import jax.numpy as jnp
from jax import lax
from jax.experimental import pallas as pl
from jax.experimental.pallas import tpu as pltpu
from jax.experimental.pallas import tpu_sc as plsc

_L = 200
_NB = 128
_G = 64


def _tc_body(ts_ref, tw_ref, pos_ref, out_ref):
    ts = ts_ref[...].astype(jnp.float32)
    nxt = jnp.concatenate([ts[:, 1:], ts[:, _L - 1 : _L]], axis=1)
    diff = nxt[:, :, None] - ts[:, None, :]
    m = jnp.maximum(jnp.abs(diff), 1.0)
    b = (jnp.log(m) * jnp.float32(1.0 / 0.301)).astype(jnp.int32)
    g = ts.shape[0]
    table = jnp.broadcast_to(tw_ref[0][None, None, :], (g, _L, _NB))
    w = jnp.take_along_axis(table, b, axis=-1, mode="promise_in_bounds")
    out_ref[...] = w + pos_ref[...]


def _tc_call(timestamps, tw, pos, g):
    Bsz = timestamps.shape[0]
    return pl.pallas_call(
        _tc_body,
        grid=(Bsz // g,),
        in_specs=[
            pl.BlockSpec((g, _L), lambda i: (i, 0)),
            pl.BlockSpec((1, _NB), lambda i: (0, 0)),
            pl.BlockSpec((_L, _L), lambda i: (0, 0)),
        ],
        out_specs=pl.BlockSpec((g, _L, _L), lambda i: (i, 0, 0)),
        out_shape=jax.ShapeDtypeStruct((Bsz, _L, _L), jnp.float32),
    )(timestamps, tw, pos)


def _f32_bucket(m):
    mf = np.maximum(np.float32(m), np.float32(1.0))
    return int(np.log(mf, dtype=np.float32) / np.float32(0.301))


def _build_tables():
    ms = []
    k = 1
    while True:
        m = int(np.ceil(np.exp(0.301 * k)))
        if m > 2**24:
            break
        lo = m - 8
        while _f32_bucket(lo) >= k:
            lo -= 8
        while _f32_bucket(lo) < k:
            lo += 1
        ms.append(lo)
        k += 1
    ms = np.array(ms)
    tbl = 640
    pad = np.float32(2**29)
    b1 = np.full(tbl, pad, np.float32)
    b2 = np.full(tbl, pad, np.float32)
    b3 = np.full(tbl, pad, np.float32)
    w4idx = np.zeros(tbl, np.int64)
    for eo in range(0, 25):
        lo, hi = 2**eo, 2 ** (eo + 1)
        base = int((ms <= lo).sum())
        bnds = ms[(ms > lo) & (ms <= hi)]
        e4 = (127 + eo) * 4
        for c in range(4):
            w4idx[e4 + c] = min(base + c, _NB)
        for j, bnd in enumerate(bnds):
            [b1, b2, b3][j][e4] = np.float32(bnd)
    bits = lambda a: a.view(np.int32)
    return bits(b1), bits(b2), bits(b3), w4idx.astype(np.int32)


_B1_BITS, _B2_BITS, _B3_BITS, _W4IDX = _build_tables()

_NC = 2
_NS = 16
_NW = _NC * _NS
_J0S = tuple(range(0, 192, 16)) + (184,)


def _sc_body(ts_hbm, w4_hbm, b1_hbm, b2_hbm, b3_hbm, pw_hbm, out_hbm,
             w4_v, b1_v, b2_v, b3_v, pw_v, ts_v, tsf_v, out_v):
    wid = lax.axis_index("s") * _NC + lax.axis_index("c")
    nbatch = ts_hbm.shape[0] // _NW
    pltpu.sync_copy(w4_hbm, w4_v)
    pltpu.sync_copy(b1_hbm, b1_v)
    pltpu.sync_copy(b2_hbm, b2_v)
    pltpu.sync_copy(b3_hbm, b3_v)
    pltpu.sync_copy(pw_hbm, pw_v)

    def batch_body(bi, carry):
        b = wid * nbatch + bi
        pltpu.sync_copy(ts_hbm.at[b], ts_v)
        for j0 in _J0S:
            tsf_v[pl.ds(j0, 16)] = ts_v[pl.ds(j0, 16)].astype(jnp.float32)

        def row_body(i, carry2):
            nxt = tsf_v[pl.ds(jnp.minimum(i + 1, _L - 1), 16)][0]
            out_row = out_v.at[i]
            for j0 in _J0S:
                tsv = tsf_v[pl.ds(j0, 16)]
                m = jnp.maximum(jnp.abs(nxt - tsv), 1.0)
                u = lax.bitcast_convert_type(m, jnp.int32)
                e4 = lax.shift_right_logical(u, 21) & (-4)
                bb1 = plsc.load_gather(b1_v, [e4])
                bb2 = plsc.load_gather(b2_v, [e4])
                bb3 = plsc.load_gather(b3_v, [e4])
                cc = (
                    jnp.where(u >= bb1, 1, 0)
                    + jnp.where(u >= bb2, 1, 0)
                    + jnp.where(u >= bb3, 1, 0)
                )
                w = plsc.load_gather(w4_v, [e4 + cc])
                pv = pw_v[pl.ds(_L - 1 - i + j0, 16)]
                out_row[pl.ds(j0, 16)] = w + pv
            return carry2

        lax.fori_loop(0, _L, row_body, 0)
        pltpu.sync_copy(out_v, out_hbm.at[b])
        return carry

    lax.fori_loop(0, nbatch, batch_body, 0)


def _sc_call(timestamps, w4, b1, b2, b3, pw_pad):
    Bsz = timestamps.shape[0]
    mesh = plsc.VectorSubcoreMesh(core_axis_name="c", subcore_axis_name="s")
    fn = functools.partial(
        pl.kernel,
        mesh=mesh,
        compiler_params=pltpu.CompilerParams(
            use_tc_tiling_on_sc=False, needs_layout_passes=False
        ),
        out_type=jax.ShapeDtypeStruct((Bsz, _L, _L), jnp.float32),
        scratch_types=[
            pltpu.VMEM((640,), jnp.float32),
            pltpu.VMEM((640,), jnp.int32),
            pltpu.VMEM((640,), jnp.int32),
            pltpu.VMEM((640,), jnp.int32),
            pltpu.VMEM((408,), jnp.float32),
            pltpu.VMEM((_L,), jnp.int32),
            pltpu.VMEM((216,), jnp.float32),
            pltpu.VMEM((_L, _L), jnp.float32),
        ],
    )(_sc_body)
    return fn(timestamps, w4, b1, b2, b3, pw_pad)


def kernel(timestamps, timestamp_weights, position_weights):
    t = jnp.concatenate(
        [position_weights[: 2 * _L - 1], jnp.zeros((_L,), dtype=position_weights.dtype)]
    )
    t = jnp.tile(t, _L)[: -_L].reshape(_L, 3 * _L - 2)
    r = (2 * _L - 1) // 2
    pos = t[:, r : 3 * _L - 2 - r]

    w4 = jnp.take(timestamp_weights, jnp.asarray(_W4IDX), axis=0)
    pw_pad = jnp.concatenate(
        [position_weights, jnp.zeros((408 - (2 * _L - 1),), jnp.float32)]
    )
    sc_out = _sc_call(
        timestamps, w4, jnp.asarray(_B1_BITS), jnp.asarray(_B2_BITS),
        jnp.asarray(_B3_BITS), pw_pad,
    )
    return sc_out


# speedup vs baseline: 4.0943x; 4.0943x over previous
import functools

import numpy as np

import jax
import jax.numpy as jnp
from jax import lax
from jax.experimental import pallas as pl
from jax.experimental.pallas import tpu as pltpu
from jax.experimental.pallas import tpu_sc as plsc

_L = 200
_NB = 128
_G = 64
_SC_BATCHES = 128


def _tc_body(ts_ref, tw_ref, pos_ref, out_ref):
    ts = ts_ref[...].astype(jnp.float32)
    nxt = jnp.concatenate([ts[:, 1:], ts[:, _L - 1 : _L]], axis=1)
    diff = nxt[:, :, None] - ts[:, None, :]
    m = jnp.maximum(jnp.abs(diff), 1.0)
    b = (jnp.log(m) * jnp.float32(1.0 / 0.301)).astype(jnp.int32)
    g = ts.shape[0]
    table = jnp.broadcast_to(tw_ref[0][None, None, :], (g, _L, _NB))
    w = jnp.take_along_axis(table, b, axis=-1, mode="promise_in_bounds")
    out_ref[...] = w + pos_ref[...]


def _tc_call(timestamps, tw, pos, g):
    Bsz = timestamps.shape[0]
    return pl.pallas_call(
        _tc_body,
        grid=(Bsz // g,),
        in_specs=[
            pl.BlockSpec((g, _L), lambda i: (i, 0)),
            pl.BlockSpec((1, _NB), lambda i: (0, 0)),
            pl.BlockSpec((_L, _L), lambda i: (0, 0)),
        ],
        out_specs=pl.BlockSpec((g, _L, _L), lambda i: (i, 0, 0)),
        out_shape=jax.ShapeDtypeStruct((Bsz, _L, _L), jnp.float32),
    )(timestamps, tw, pos)


def _f32_bucket(m):
    mf = np.maximum(np.float32(m), np.float32(1.0))
    return int(np.log(mf, dtype=np.float32) / np.float32(0.301))


def _build_tables():
    ms = []
    k = 1
    while True:
        m = int(np.ceil(np.exp(0.301 * k)))
        if m > 2**24:
            break
        lo = m - 8
        while _f32_bucket(lo) >= k:
            lo -= 8
        while _f32_bucket(lo) < k:
            lo += 1
        ms.append(lo)
        k += 1
    ms = np.array(ms)
    tbl = 640
    pad = np.float32(2**29)
    b1 = np.full(tbl, pad, np.float32)
    b2 = np.full(tbl, pad, np.float32)
    b3 = np.full(tbl, pad, np.float32)
    w4idx = np.zeros(tbl, np.int64)
    for eo in range(0, 25):
        lo, hi = 2**eo, 2 ** (eo + 1)
        base = int((ms <= lo).sum())
        bnds = ms[(ms > lo) & (ms <= hi)]
        e4 = (127 + eo) * 4
        for c in range(4):
            w4idx[e4 + c] = min(base + c, _NB)
        for j, bnd in enumerate(bnds):
            [b1, b2, b3][j][e4] = np.float32(bnd)
    bits = lambda a: a.view(np.int32)
    return bits(b1), bits(b2), bits(b3), w4idx.astype(np.int32)


_B1_BITS, _B2_BITS, _B3_BITS, _W4IDX = _build_tables()

_NC = 2
_NS = 16
_NW = _NC * _NS
_J0S = tuple(range(0, 192, 16)) + (184,)


def _sc_body(ts_hbm, w4_hbm, b1_hbm, b2_hbm, b3_hbm, pw_hbm, out_hbm,
             w4_v, b1_v, b2_v, b3_v, pw_v, ts_v, tsf_v, out_v):
    wid = lax.axis_index("s") * _NC + lax.axis_index("c")
    nbatch = ts_hbm.shape[0] // _NW
    pltpu.sync_copy(w4_hbm, w4_v)
    pltpu.sync_copy(b1_hbm, b1_v)
    pltpu.sync_copy(b2_hbm, b2_v)
    pltpu.sync_copy(b3_hbm, b3_v)
    pltpu.sync_copy(pw_hbm, pw_v)

    def batch_body(bi, carry):
        b = wid * nbatch + bi
        pltpu.sync_copy(ts_hbm.at[b], ts_v)
        for j0 in _J0S:
            tsf_v[pl.ds(j0, 16)] = ts_v[pl.ds(j0, 16)].astype(jnp.float32)

        def row_body(i, carry2):
            nxt = tsf_v[pl.ds(jnp.minimum(i + 1, _L - 1), 16)][0]
            out_row = out_v.at[i]
            for j0 in _J0S:
                tsv = tsf_v[pl.ds(j0, 16)]
                m = jnp.maximum(jnp.abs(nxt - tsv), 1.0)
                u = lax.bitcast_convert_type(m, jnp.int32)
                e4 = lax.shift_right_logical(u, 21) & (-4)
                bb1 = plsc.load_gather(b1_v, [e4])
                bb2 = plsc.load_gather(b2_v, [e4])
                bb3 = plsc.load_gather(b3_v, [e4])
                cc = (
                    jnp.where(u >= bb1, 1, 0)
                    + jnp.where(u >= bb2, 1, 0)
                    + jnp.where(u >= bb3, 1, 0)
                )
                w = plsc.load_gather(w4_v, [e4 + cc])
                pv = pw_v[pl.ds(_L - 1 - i + j0, 16)]
                out_row[pl.ds(j0, 16)] = w + pv
            return carry2

        lax.fori_loop(0, _L, row_body, 0)
        pltpu.sync_copy(out_v, out_hbm.at[b])
        return carry

    lax.fori_loop(0, nbatch, batch_body, 0)


def _sc_call(timestamps, w4, b1, b2, b3, pw_pad):
    Bsz = timestamps.shape[0]
    mesh = plsc.VectorSubcoreMesh(core_axis_name="c", subcore_axis_name="s")
    fn = functools.partial(
        pl.kernel,
        mesh=mesh,
        compiler_params=pltpu.CompilerParams(
            use_tc_tiling_on_sc=False, needs_layout_passes=False
        ),
        out_type=jax.ShapeDtypeStruct((Bsz, _L, _L), jnp.float32),
        scratch_types=[
            pltpu.VMEM((640,), jnp.float32),
            pltpu.VMEM((640,), jnp.int32),
            pltpu.VMEM((640,), jnp.int32),
            pltpu.VMEM((640,), jnp.int32),
            pltpu.VMEM((408,), jnp.float32),
            pltpu.VMEM((_L,), jnp.int32),
            pltpu.VMEM((216,), jnp.float32),
            pltpu.VMEM((_L, _L), jnp.float32),
        ],
    )(_sc_body)
    return fn(timestamps, w4, b1, b2, b3, pw_pad)


def kernel(timestamps, timestamp_weights, position_weights):
    t = jnp.concatenate(
        [position_weights[: 2 * _L - 1], jnp.zeros((_L,), dtype=position_weights.dtype)]
    )
    t = jnp.tile(t, _L)[: -_L].reshape(_L, 3 * _L - 2)
    r = (2 * _L - 1) // 2
    pos = t[:, r : 3 * _L - 2 - r]

    w4 = jnp.take(timestamp_weights, jnp.asarray(_W4IDX), axis=0)
    pw_pad = jnp.concatenate(
        [position_weights, jnp.zeros((408 - (2 * _L - 1),), jnp.float32)]
    )
    tw = timestamp_weights[:_NB].reshape(1, _NB)

    s = _SC_BATCHES
    sc_out = _sc_call(
        timestamps[:s], w4, jnp.asarray(_B1_BITS), jnp.asarray(_B2_BITS),
        jnp.asarray(_B3_BITS), pw_pad,
    )
    tc_out = _tc_call(timestamps[s:], tw, pos, _G)
    return jnp.concatenate([sc_out, tc_out], axis=0)


# speedup vs baseline: 6.4678x; 1.5797x over previous
import functools

import numpy as np

import jax
import jax.numpy as jnp
from jax import lax
from jax.experimental import pallas as pl
from jax.experimental.pallas import tpu as pltpu
from jax.experimental.pallas import tpu_sc as plsc

_L = 200
_NB = 128
_G = 64
_SC_BATCHES = 64


def _tc_body(ts_ref, tw_ref, pos_ref, out_ref):
    ts = ts_ref[...].astype(jnp.float32)
    nxt = jnp.concatenate([ts[:, 1:], ts[:, _L - 1 : _L]], axis=1)
    diff = nxt[:, :, None] - ts[:, None, :]
    m = jnp.maximum(jnp.abs(diff), 1.0)
    b = (jnp.log(m) * jnp.float32(1.0 / 0.301)).astype(jnp.int32)
    g = ts.shape[0]
    table = jnp.broadcast_to(tw_ref[0][None, None, :], (g, _L, _NB))
    w = jnp.take_along_axis(table, b, axis=-1, mode="promise_in_bounds")
    out_ref[...] = w + pos_ref[...]


def _tc_call(timestamps, tw, pos, g, skip_blocks=0):
    Bsz = timestamps.shape[0]
    return pl.pallas_call(
        _tc_body,
        grid=(Bsz // g - skip_blocks,),
        in_specs=[
            pl.BlockSpec((g, _L), lambda i: (i + skip_blocks, 0)),
            pl.BlockSpec((1, _NB), lambda i: (0, 0)),
            pl.BlockSpec((_L, _L), lambda i: (0, 0)),
        ],
        out_specs=pl.BlockSpec((g, _L, _L), lambda i: (i + skip_blocks, 0, 0)),
        out_shape=jax.ShapeDtypeStruct((Bsz, _L, _L), jnp.float32),
    )(timestamps, tw, pos)


def _f32_bucket(m):
    mf = np.maximum(np.float32(m), np.float32(1.0))
    return int(np.log(mf, dtype=np.float32) / np.float32(0.301))


def _build_tables():
    ms = []
    k = 1
    while True:
        m = int(np.ceil(np.exp(0.301 * k)))
        if m > 2**24:
            break
        lo = m - 8
        while _f32_bucket(lo) >= k:
            lo -= 8
        while _f32_bucket(lo) < k:
            lo += 1
        ms.append(lo)
        k += 1
    ms = np.array(ms)
    tbl = 640
    pad = np.float32(2**29)
    b1 = np.full(tbl, pad, np.float32)
    b2 = np.full(tbl, pad, np.float32)
    b3 = np.full(tbl, pad, np.float32)
    w4idx = np.zeros(tbl, np.int64)
    for eo in range(0, 25):
        lo, hi = 2**eo, 2 ** (eo + 1)
        base = int((ms <= lo).sum())
        bnds = ms[(ms > lo) & (ms <= hi)]
        e4 = (127 + eo) * 4
        for c in range(4):
            w4idx[e4 + c] = min(base + c, _NB)
        for j, bnd in enumerate(bnds):
            [b1, b2, b3][j][e4] = np.float32(bnd)
    bits = lambda a: a.view(np.int32)
    return bits(b1), bits(b2), bits(b3), w4idx.astype(np.int32)


_B1_BITS, _B2_BITS, _B3_BITS, _W4IDX = _build_tables()

_NC = 2
_NS = 16
_NW = _NC * _NS
_J0S = tuple(range(0, 192, 16)) + (184,)


def _sc_body(ts_hbm, w4_hbm, b1_hbm, b2_hbm, b3_hbm, pw_hbm, out_hbm,
             w4_v, b1_v, b2_v, b3_v, pw_v, ts_v, tsf_v, out_v):
    wid = lax.axis_index("s") * _NC + lax.axis_index("c")
    nbatch = ts_hbm.shape[0] // _NW
    pltpu.sync_copy(w4_hbm, w4_v)
    pltpu.sync_copy(b1_hbm, b1_v)
    pltpu.sync_copy(b2_hbm, b2_v)
    pltpu.sync_copy(b3_hbm, b3_v)
    pltpu.sync_copy(pw_hbm, pw_v)

    def batch_body(bi, carry):
        b = wid * nbatch + bi
        pltpu.sync_copy(ts_hbm.at[b], ts_v)
        for j0 in _J0S:
            tsf_v[pl.ds(j0, 16)] = ts_v[pl.ds(j0, 16)].astype(jnp.float32)

        def row_body(i, carry2):
            nxt = tsf_v[pl.ds(jnp.minimum(i + 1, _L - 1), 16)][0]
            out_row = out_v.at[i]
            for j0 in _J0S:
                tsv = tsf_v[pl.ds(j0, 16)]
                m = jnp.maximum(jnp.abs(nxt - tsv), 1.0)
                u = lax.bitcast_convert_type(m, jnp.int32)
                e4 = lax.shift_right_logical(u, 21) & (-4)
                bb1 = plsc.load_gather(b1_v, [e4])
                bb2 = plsc.load_gather(b2_v, [e4])
                bb3 = plsc.load_gather(b3_v, [e4])
                cc = (
                    jnp.where(u >= bb1, 1, 0)
                    + jnp.where(u >= bb2, 1, 0)
                    + jnp.where(u >= bb3, 1, 0)
                )
                w = plsc.load_gather(w4_v, [e4 + cc])
                pv = pw_v[pl.ds(_L - 1 - i + j0, 16)]
                out_row[pl.ds(j0, 16)] = w + pv
            return carry2

        lax.fori_loop(0, _L, row_body, 0)
        pltpu.sync_copy(out_v, out_hbm.at[b])
        return carry

    lax.fori_loop(0, nbatch, batch_body, 0)


def _sc_call(timestamps, w4, b1, b2, b3, pw_pad):
    Bsz = timestamps.shape[0]
    mesh = plsc.VectorSubcoreMesh(core_axis_name="c", subcore_axis_name="s")
    fn = functools.partial(
        pl.kernel,
        mesh=mesh,
        compiler_params=pltpu.CompilerParams(
            use_tc_tiling_on_sc=False, needs_layout_passes=False
        ),
        out_type=jax.ShapeDtypeStruct((Bsz, _L, _L), jnp.float32),
        scratch_types=[
            pltpu.VMEM((640,), jnp.float32),
            pltpu.VMEM((640,), jnp.int32),
            pltpu.VMEM((640,), jnp.int32),
            pltpu.VMEM((640,), jnp.int32),
            pltpu.VMEM((408,), jnp.float32),
            pltpu.VMEM((_L,), jnp.int32),
            pltpu.VMEM((216,), jnp.float32),
            pltpu.VMEM((_L, _L), jnp.float32),
        ],
    )(_sc_body)
    return fn(timestamps, w4, b1, b2, b3, pw_pad)


def kernel(timestamps, timestamp_weights, position_weights):
    t = jnp.concatenate(
        [position_weights[: 2 * _L - 1], jnp.zeros((_L,), dtype=position_weights.dtype)]
    )
    t = jnp.tile(t, _L)[: -_L].reshape(_L, 3 * _L - 2)
    r = (2 * _L - 1) // 2
    pos = t[:, r : 3 * _L - 2 - r]

    w4 = jnp.take(timestamp_weights, jnp.asarray(_W4IDX), axis=0)
    pw_pad = jnp.concatenate(
        [position_weights, jnp.zeros((408 - (2 * _L - 1),), jnp.float32)]
    )
    tw = timestamp_weights[:_NB].reshape(1, _NB)

    s = _SC_BATCHES
    sc_out = _sc_call(
        timestamps[:s], w4, jnp.asarray(_B1_BITS), jnp.asarray(_B2_BITS),
        jnp.asarray(_B3_BITS), pw_pad,
    )
    tc_full = _tc_call(timestamps, tw, pos, _G, skip_blocks=s // _G)
    return lax.dynamic_update_slice(tc_full, sc_out, (0, 0, 0))
